# Initial kernel scaffold; baseline (speedup 1.0000x reference)
import jax
import jax.numpy as jnp
from jax.experimental import pallas as pl


def kernel(x, edge_index, batch, params):
    raise NotImplementedError("write your pallas kernel here")


# speedup vs baseline: 14.5744x; 14.5744x over previous
import functools

import numpy as np
import jax
import jax.numpy as jnp
from jax import lax
from jax.experimental import pallas as pl
from jax.experimental.pallas import tpu as pltpu
from jax.experimental.pallas import tpu_sc as plsc

_NG = 100
_F = 128
_N = _NG * _F
_E = 409600
_HID = 64
_NC, _NS, _L = 2, 16, 16
_NW = _NC * _NS
_EW = _E // _NW
_CH = 128
_NCHUNK = _EW // _CH
_RPS = _N // _NS

@functools.cache
def _sc_mesh():
    return plsc.VectorSubcoreMesh(core_axis_name="c", subcore_axis_name="s",
                                  num_cores=_NC, num_subcores=_NS)

_IU = np.triu_indices(_F, 1)
_TRIU_FLAT = (_IU[0] * _F + _IU[1]).astype(np.int32)
_INV = np.zeros((_F * _F,), np.int32)
_INV[_TRIU_FLAT] = np.arange(_TRIU_FLAT.size, dtype=np.int32)
_TRIU_MASK = np.zeros((_F * _F, 1), np.float32)
_TRIU_MASK[_TRIU_FLAT, 0] = 1.0


def _deg_body(dst_hbm, out_hbm, idx_v, hist_v):
    c = lax.axis_index("c")
    s = lax.axis_index("s")
    wid = s * _NC + c
    pltpu.sync_copy(dst_hbm.at[pl.ds(wid * _EW, _EW)], idx_v)
    zero16 = jnp.zeros((_L,), jnp.float32)
    one16 = jnp.ones((_L,), jnp.float32)

    def zbody(i, carry):
        hist_v[pl.ds(i * _L, _L)] = zero16
        return carry

    lax.fori_loop(0, _N // _L, zbody, 0)

    def hbody(i, carry):
        idx = idx_v[pl.ds(i * _L, _L)]
        plsc.addupdate_scatter(hist_v, [idx], one16)
        return carry

    lax.fori_loop(0, _EW // _L, hbody, 0)
    pltpu.sync_copy(hist_v, out_hbm.at[wid])


@functools.cache
def _deg_call():
    return pl.kernel(
        _deg_body,
        out_type=jax.ShapeDtypeStruct((_NW, _N), jnp.float32),
        mesh=_sc_mesh(),
        scratch_types=[
            pltpu.VMEM((_EW,), jnp.int32),
            pltpu.VMEM((_N,), jnp.float32),
        ],
        compiler_params=pltpu.CompilerParams(needs_layout_passes=False),
    )


def _scat_body(tab_hbm, src_hbm, dst_hbm, zero_hbm, out_hbm,
               sidx_v, didx_v, row_v, acc_sh):
    c = lax.axis_index("c")
    s = lax.axis_index("s")
    wid = s * _NC + c
    pltpu.sync_copy(zero_hbm.at[pl.ds(s * _RPS, _RPS)],
                    acc_sh.at[pl.ds(s * _RPS, _RPS)])
    pltpu.sync_copy(src_hbm.at[pl.ds(wid * _NCHUNK, _NCHUNK)], sidx_v)
    pltpu.sync_copy(dst_hbm.at[pl.ds(wid * _NCHUNK, _NCHUNK)], didx_v)
    plsc.subcore_barrier()

    def body(j, carry):
        pltpu.sync_copy(tab_hbm.at[sidx_v.at[j]], row_v)
        pltpu.sync_copy(row_v, acc_sh.at[didx_v.at[j]], add=True)
        return carry

    lax.fori_loop(0, _NCHUNK, body, 0)
    plsc.subcore_barrier()
    pltpu.sync_copy(acc_sh.at[pl.ds(s * _RPS, _RPS)],
                    out_hbm.at[c, pl.ds(s * _RPS, _RPS)])


@functools.cache
def _scat_call():
    return pl.kernel(
        _scat_body,
        out_type=jax.ShapeDtypeStruct((_NC, _N, _HID), jnp.float32),
        mesh=_sc_mesh(),
        scratch_types=[
            pltpu.VMEM((_NCHUNK, _CH), jnp.int32),
            pltpu.VMEM((_NCHUNK, _CH), jnp.int32),
            pltpu.VMEM((_CH, _HID), jnp.float32),
            pltpu.VMEM_SHARED((_N, _HID), jnp.float32),
        ],
        compiler_params=pltpu.CompilerParams(needs_layout_passes=False,
                                             use_tc_tiling_on_sc=False),
    )


def _ga_body(x_ref, w0_ref, d32_ref, g0_ref, dinv_ref):
    deg = jnp.sum(d32_ref[...], axis=1, keepdims=True) + 1.0
    dinv = lax.rsqrt(jnp.maximum(deg, 1.0))
    hw = jnp.dot(x_ref[...], w0_ref[...], preferred_element_type=jnp.float32)
    g0_ref[...] = hw * dinv
    dinv_ref[...] = dinv


def _ga(x, w0, d32t):
    return pl.pallas_call(
        _ga_body,
        out_shape=(
            jax.ShapeDtypeStruct((_N, _HID), jnp.float32),
            jax.ShapeDtypeStruct((_N, 1), jnp.float32),
        ),
    )(x, w0, d32t)


def _c_body(m_ref, g0_ref, dinv_ref, b0_ref, w1_ref, h1_ref, g1_ref):
    dinv = dinv_ref[...]
    h1 = jnp.tanh((m_ref[0] + m_ref[1] + g0_ref[...]) * dinv + b0_ref[...])
    h1_ref[...] = h1
    g1_ref[...] = jnp.dot(h1, w1_ref[...],
                          preferred_element_type=jnp.float32) * dinv


def _c(m0, g0, dinv, b0, w1):
    return pl.pallas_call(
        _c_body,
        out_shape=(
            jax.ShapeDtypeStruct((_N, _HID), jnp.float32),
            jax.ShapeDtypeStruct((_N, _HID), jnp.float32),
        ),
    )(m0, g0, dinv, b0, w1)


def _d1_body(m_ref, g1_ref, dinv_ref, b1_ref, h1_ref, bg_ref, bb_ref, out_ref):
    dinv = dinv_ref[...]
    h2 = jnp.tanh((m_ref[0] + m_ref[1] + g1_ref[...]) * dinv + b1_ref[...])
    hcat = jnp.concatenate([h1_ref[...], h2], axis=1)
    rr = lax.broadcasted_iota(jnp.int32, (_NG, _N), 0)
    cc = lax.broadcasted_iota(jnp.int32, (_NG, _N), 1)
    sel = jnp.where((cc // _F) == rr, jnp.float32(1.0 / _F), jnp.float32(0.0))
    m = jnp.dot(sel, hcat, preferred_element_type=jnp.float32)
    mu = jnp.mean(m, axis=0, keepdims=True)
    var = jnp.mean((m - mu) ** 2, axis=0, keepdims=True)
    out_ref[...] = (m - mu) * lax.rsqrt(var + 1e-5) * bg_ref[...] + bb_ref[...]


def _d1(m1, g1, dinv, b1, h1, bg, bb):
    return pl.pallas_call(
        _d1_body,
        out_shape=jax.ShapeDtypeStruct((_NG, _F), jnp.float32),
    )(m1, g1, dinv, b1, h1, bg, bb)


def _bn_relu(z, g, b):
    mu = jnp.mean(z, axis=0, keepdims=True)
    var = jnp.mean((z - mu) ** 2, axis=0, keepdims=True)
    return jnp.maximum((z - mu) * lax.rsqrt(var + 1e-5) * g + b,
                       jnp.float32(0.0))


def _d2_body(xf_ref, wfull_ref, gfull_ref, bfull_ref, hbn_ref, wh_ref,
             b0_ref, g0_ref, bb0_ref, w1_ref, b1_ref, g1_ref, bb1_ref,
             w2_ref, b2_ref, g2_ref, bb2_ref, w3_ref, b3_ref, out_ref):
    xf = xf_ref[...]
    mu = jnp.mean(xf, axis=0, keepdims=True)
    var = jnp.mean((xf - mu) ** 2, axis=0, keepdims=True)
    xbn = (xf - mu) * lax.rsqrt(var + 1e-5) * gfull_ref[...] + bfull_ref[...]
    z = (jnp.dot(xbn, wfull_ref[...], preferred_element_type=jnp.float32)
         + jnp.dot(hbn_ref[...], wh_ref[...],
                   preferred_element_type=jnp.float32)
         + b0_ref[...])
    z = _bn_relu(z, g0_ref[...], bb0_ref[...])
    z = _bn_relu(jnp.dot(z, w1_ref[...], preferred_element_type=jnp.float32)
                 + b1_ref[...], g1_ref[...], bb1_ref[...])
    z = _bn_relu(jnp.dot(z, w2_ref[...], preferred_element_type=jnp.float32)
                 + b2_ref[...], g2_ref[...], bb2_ref[...])
    out_ref[...] = (jnp.dot(z, w3_ref[...], preferred_element_type=jnp.float32)
                    + b3_ref[...])


def _d2(xf, wfull, gfull, bfull, hbn, wh, p):
    return pl.pallas_call(
        _d2_body,
        out_shape=jax.ShapeDtypeStruct((_NG, 2), jnp.float32),
    )(xf, wfull, gfull, bfull, hbn, wh,
      p["mlp0_b"].reshape(1, -1), p["mbn0_g"].reshape(1, -1),
      p["mbn0_b"].reshape(1, -1),
      p["mlp1_W"], p["mlp1_b"].reshape(1, -1), p["mbn1_g"].reshape(1, -1),
      p["mbn1_b"].reshape(1, -1),
      p["mlp2_W"], p["mlp2_b"].reshape(1, -1), p["mbn2_g"].reshape(1, -1),
      p["mbn2_b"].reshape(1, -1),
      p["mlp3_W"], p["mlp3_b"].reshape(1, -1))


def kernel(x, edge_index, batch, params):
    del batch
    p = params
    src_flat = edge_index[0]
    dst_flat = edge_index[1]
    src_r = src_flat.reshape(_NW * _NCHUNK, _CH)
    dst_r = dst_flat.reshape(_NW * _NCHUNK, _CH)
    zeros_tab = jnp.zeros((_N, _HID), jnp.float32)

    d32 = _deg_call()(dst_flat)
    g0, dinv = _ga(x, p["conv0_W"], d32.T)
    m0 = _scat_call()(g0, src_r, dst_r, zeros_tab)
    h1, g1 = _c(m0, g0, dinv, p["conv0_b"].reshape(1, -1), p["conv1_W"])
    m1 = _scat_call()(g1, src_r, dst_r, zeros_tab)
    hbn = _d1(m1, g1, dinv, p["conv1_b"].reshape(1, -1), h1,
              p["bnh_g"].reshape(1, -1), p["bnh_b"].reshape(1, -1))

    inv = jnp.asarray(_INV)
    wfull = p["mlp0_W"][: _TRIU_FLAT.size][inv] * jnp.asarray(_TRIU_MASK)
    gfull = p["bn_g"][inv].reshape(1, -1)
    bfull = p["bn_b"][inv].reshape(1, -1)
    xf = x.reshape(_NG, _F * _F)
    wh = p["mlp0_W"][_TRIU_FLAT.size:]
    return _d2(xf, wfull, gfull, bfull, hbn, wh, p)


# speedup vs baseline: 15.2692x; 1.0477x over previous
import functools

import numpy as np
import jax
import jax.numpy as jnp
from jax import lax
from jax.experimental import pallas as pl
from jax.experimental.pallas import tpu as pltpu
from jax.experimental.pallas import tpu_sc as plsc

_NG = 100
_F = 128
_N = _NG * _F
_E = 409600
_HID = 64
_NC, _NS, _L = 2, 16, 16
_NW = _NC * _NS
_EW = _E // _NW
_CH = 128
_NCHUNK = _EW // _CH
_RPS = _N // _NS

@functools.cache
def _sc_mesh():
    return plsc.VectorSubcoreMesh(core_axis_name="c", subcore_axis_name="s",
                                  num_cores=_NC, num_subcores=_NS)

_IU = np.triu_indices(_F, 1)
_TRIU_FLAT = (_IU[0] * _F + _IU[1]).astype(np.int32)
_INV = np.zeros((_F * _F,), np.int32)
_INV[_TRIU_FLAT] = np.arange(_TRIU_FLAT.size, dtype=np.int32)
_TRIU_MASK = np.zeros((_F * _F, 1), np.float32)
_TRIU_MASK[_TRIU_FLAT, 0] = 1.0


def _deg_body(dst_hbm, out_hbm, idx_v, hist_v):
    c = lax.axis_index("c")
    s = lax.axis_index("s")
    wid = s * _NC + c
    pltpu.sync_copy(dst_hbm.at[pl.ds(wid * _EW, _EW)], idx_v)
    zero16 = jnp.zeros((_L,), jnp.float32)
    one16 = jnp.ones((_L,), jnp.float32)

    def zbody(i, carry):
        hist_v[pl.ds(i * _L, _L)] = zero16
        return carry

    lax.fori_loop(0, _N // _L, zbody, 0)

    def hbody(i, carry):
        idx = idx_v[pl.ds(i * _L, _L)]
        plsc.addupdate_scatter(hist_v, [idx], one16)
        return carry

    lax.fori_loop(0, _EW // _L, hbody, 0)
    pltpu.sync_copy(hist_v, out_hbm.at[wid])


@functools.cache
def _deg_call():
    return pl.kernel(
        _deg_body,
        out_type=jax.ShapeDtypeStruct((_NW, _N), jnp.float32),
        mesh=_sc_mesh(),
        scratch_types=[
            pltpu.VMEM((_EW,), jnp.int32),
            pltpu.VMEM((_N,), jnp.float32),
        ],
        compiler_params=pltpu.CompilerParams(needs_layout_passes=False),
    )


def _scat_body(tab_hbm, src_hbm, dst_hbm, zero_hbm, out_hbm,
               sidx_v, didx_v, row_v, acc_sh,
               gsem_a, gsem_b, ssem_a, ssem_b):
    c = lax.axis_index("c")
    s = lax.axis_index("s")
    wid = s * _NC + c
    pltpu.sync_copy(zero_hbm.at[pl.ds(s * _RPS, _RPS)],
                    acc_sh.at[pl.ds(s * _RPS, _RPS)])
    pltpu.sync_copy(src_hbm.at[pl.ds(wid * _NCHUNK, _NCHUNK)], sidx_v)
    pltpu.sync_copy(dst_hbm.at[pl.ds(wid * _NCHUNK, _NCHUNK)], didx_v)
    plsc.subcore_barrier()

    row_a = row_v.at[0]
    row_b = row_v.at[1]

    def _gather(j, row, sem):
        pltpu.async_copy(tab_hbm.at[sidx_v.at[j]], row, sem)

    def _gwait(j, row, sem):
        pltpu.make_async_copy(tab_hbm.at[sidx_v.at[j]], row, sem).wait()

    def _scat(j, row, sem):
        pltpu.async_copy(row, acc_sh.at[didx_v.at[j]], sem, add=True)

    def _swait(j, row, sem):
        pltpu.make_async_copy(row, acc_sh.at[didx_v.at[j]], sem).wait()

    _gather(0, row_a, gsem_a)

    def body(i, carry):
        ja = 2 * i
        jb = 2 * i + 1
        _gwait(ja, row_a, gsem_a)
        _scat(ja, row_a, ssem_a)

        @pl.when(i > 0)
        def _():
            _swait(jb - 2, row_b, ssem_b)

        _gather(jb, row_b, gsem_b)
        _gwait(jb, row_b, gsem_b)
        _scat(jb, row_b, ssem_b)
        _swait(ja, row_a, ssem_a)

        @pl.when(i < _NCHUNK // 2 - 1)
        def _():
            _gather(ja + 2, row_a, gsem_a)

        return carry

    lax.fori_loop(0, _NCHUNK // 2, body, 0)
    _swait(_NCHUNK - 1, row_b, ssem_b)
    plsc.subcore_barrier()
    pltpu.sync_copy(acc_sh.at[pl.ds(s * _RPS, _RPS)],
                    out_hbm.at[c, pl.ds(s * _RPS, _RPS)])


@functools.cache
def _scat_call():
    return pl.kernel(
        _scat_body,
        out_type=jax.ShapeDtypeStruct((_NC, _N, _HID), jnp.float32),
        mesh=_sc_mesh(),
        scratch_types=[
            pltpu.VMEM((_NCHUNK, _CH), jnp.int32),
            pltpu.VMEM((_NCHUNK, _CH), jnp.int32),
            pltpu.VMEM((2, _CH, _HID), jnp.float32),
            pltpu.VMEM_SHARED((_N, _HID), jnp.float32),
            pltpu.SemaphoreType.DMA,
            pltpu.SemaphoreType.DMA,
            pltpu.SemaphoreType.DMA,
            pltpu.SemaphoreType.DMA,
        ],
        compiler_params=pltpu.CompilerParams(needs_layout_passes=False,
                                             use_tc_tiling_on_sc=False),
    )


def _ga_body(x_ref, w0_ref, d32_ref, g0_ref, dinv_ref):
    deg = jnp.sum(d32_ref[...], axis=1, keepdims=True) + 1.0
    dinv = lax.rsqrt(jnp.maximum(deg, 1.0))
    hw = jnp.dot(x_ref[...], w0_ref[...], preferred_element_type=jnp.float32)
    g0_ref[...] = hw * dinv
    dinv_ref[...] = dinv


def _ga(x, w0, d32t):
    return pl.pallas_call(
        _ga_body,
        out_shape=(
            jax.ShapeDtypeStruct((_N, _HID), jnp.float32),
            jax.ShapeDtypeStruct((_N, 1), jnp.float32),
        ),
    )(x, w0, d32t)


def _c_body(m_ref, g0_ref, dinv_ref, b0_ref, w1_ref, h1_ref, g1_ref):
    dinv = dinv_ref[...]
    h1 = jnp.tanh((m_ref[0] + m_ref[1] + g0_ref[...]) * dinv + b0_ref[...])
    h1_ref[...] = h1
    g1_ref[...] = jnp.dot(h1, w1_ref[...],
                          preferred_element_type=jnp.float32) * dinv


def _c(m0, g0, dinv, b0, w1):
    return pl.pallas_call(
        _c_body,
        out_shape=(
            jax.ShapeDtypeStruct((_N, _HID), jnp.float32),
            jax.ShapeDtypeStruct((_N, _HID), jnp.float32),
        ),
    )(m0, g0, dinv, b0, w1)


def _d1_body(m_ref, g1_ref, dinv_ref, b1_ref, h1_ref, bg_ref, bb_ref, out_ref):
    dinv = dinv_ref[...]
    h2 = jnp.tanh((m_ref[0] + m_ref[1] + g1_ref[...]) * dinv + b1_ref[...])
    hcat = jnp.concatenate([h1_ref[...], h2], axis=1)
    rr = lax.broadcasted_iota(jnp.int32, (_NG, _N), 0)
    cc = lax.broadcasted_iota(jnp.int32, (_NG, _N), 1)
    sel = jnp.where((cc // _F) == rr, jnp.float32(1.0 / _F), jnp.float32(0.0))
    m = jnp.dot(sel, hcat, preferred_element_type=jnp.float32)
    mu = jnp.mean(m, axis=0, keepdims=True)
    var = jnp.mean((m - mu) ** 2, axis=0, keepdims=True)
    out_ref[...] = (m - mu) * lax.rsqrt(var + 1e-5) * bg_ref[...] + bb_ref[...]


def _d1(m1, g1, dinv, b1, h1, bg, bb):
    return pl.pallas_call(
        _d1_body,
        out_shape=jax.ShapeDtypeStruct((_NG, _F), jnp.float32),
    )(m1, g1, dinv, b1, h1, bg, bb)


def _bn_relu(z, g, b):
    mu = jnp.mean(z, axis=0, keepdims=True)
    var = jnp.mean((z - mu) ** 2, axis=0, keepdims=True)
    return jnp.maximum((z - mu) * lax.rsqrt(var + 1e-5) * g + b,
                       jnp.float32(0.0))


def _d2_body(xf_ref, wfull_ref, gfull_ref, bfull_ref, hbn_ref, wh_ref,
             b0_ref, g0_ref, bb0_ref, w1_ref, b1_ref, g1_ref, bb1_ref,
             w2_ref, b2_ref, g2_ref, bb2_ref, w3_ref, b3_ref, out_ref):
    xf = xf_ref[...]
    mu = jnp.mean(xf, axis=0, keepdims=True)
    var = jnp.mean((xf - mu) ** 2, axis=0, keepdims=True)
    xbn = (xf - mu) * lax.rsqrt(var + 1e-5) * gfull_ref[...] + bfull_ref[...]
    z = (jnp.dot(xbn, wfull_ref[...], preferred_element_type=jnp.float32)
         + jnp.dot(hbn_ref[...], wh_ref[...],
                   preferred_element_type=jnp.float32)
         + b0_ref[...])
    z = _bn_relu(z, g0_ref[...], bb0_ref[...])
    z = _bn_relu(jnp.dot(z, w1_ref[...], preferred_element_type=jnp.float32)
                 + b1_ref[...], g1_ref[...], bb1_ref[...])
    z = _bn_relu(jnp.dot(z, w2_ref[...], preferred_element_type=jnp.float32)
                 + b2_ref[...], g2_ref[...], bb2_ref[...])
    out_ref[...] = (jnp.dot(z, w3_ref[...], preferred_element_type=jnp.float32)
                    + b3_ref[...])


def _d2(xf, wfull, gfull, bfull, hbn, wh, p):
    return pl.pallas_call(
        _d2_body,
        out_shape=jax.ShapeDtypeStruct((_NG, 2), jnp.float32),
    )(xf, wfull, gfull, bfull, hbn, wh,
      p["mlp0_b"].reshape(1, -1), p["mbn0_g"].reshape(1, -1),
      p["mbn0_b"].reshape(1, -1),
      p["mlp1_W"], p["mlp1_b"].reshape(1, -1), p["mbn1_g"].reshape(1, -1),
      p["mbn1_b"].reshape(1, -1),
      p["mlp2_W"], p["mlp2_b"].reshape(1, -1), p["mbn2_g"].reshape(1, -1),
      p["mbn2_b"].reshape(1, -1),
      p["mlp3_W"], p["mlp3_b"].reshape(1, -1))


def kernel(x, edge_index, batch, params):
    del batch
    p = params
    src_flat = edge_index[0]
    dst_flat = edge_index[1]
    src_r = src_flat.reshape(_NW * _NCHUNK, _CH)
    dst_r = dst_flat.reshape(_NW * _NCHUNK, _CH)
    zeros_tab = jnp.zeros((_N, _HID), jnp.float32)

    d32 = _deg_call()(dst_flat)
    g0, dinv = _ga(x, p["conv0_W"], d32.T)
    m0 = _scat_call()(g0, src_r, dst_r, zeros_tab)
    h1, g1 = _c(m0, g0, dinv, p["conv0_b"].reshape(1, -1), p["conv1_W"])
    m1 = _scat_call()(g1, src_r, dst_r, zeros_tab)
    hbn = _d1(m1, g1, dinv, p["conv1_b"].reshape(1, -1), h1,
              p["bnh_g"].reshape(1, -1), p["bnh_b"].reshape(1, -1))

    inv = jnp.asarray(_INV)
    wfull = p["mlp0_W"][: _TRIU_FLAT.size][inv] * jnp.asarray(_TRIU_MASK)
    gfull = p["bn_g"][inv].reshape(1, -1)
    bfull = p["bn_b"][inv].reshape(1, -1)
    xf = x.reshape(_NG, _F * _F)
    wh = p["mlp0_W"][_TRIU_FLAT.size:]
    return _d2(xf, wfull, gfull, bfull, hbn, wh, p)


# speedup vs baseline: 28.9444x; 1.8956x over previous
import functools

import numpy as np
import jax
import jax.numpy as jnp
from jax import lax
from jax.experimental import pallas as pl
from jax.experimental.pallas import tpu as pltpu
from jax.experimental.pallas import tpu_sc as plsc

_NG = 100
_F = 128
_N = _NG * _F
_E = 409600
_HID = 64
_NC, _NS, _L = 2, 16, 16
_NW = _NC * _NS
_EW = _E // _NW
_CH = 128
_NCHUNK = _EW // _CH
_RPS = _N // _NS

@functools.cache
def _sc_mesh():
    return plsc.VectorSubcoreMesh(core_axis_name="c", subcore_axis_name="s",
                                  num_cores=_NC, num_subcores=_NS)

_IU = np.triu_indices(_F, 1)
_TRIU_FLAT = (_IU[0] * _F + _IU[1]).astype(np.int32)
_TRI = _TRIU_FLAT.size
_GPW = -(-_NG // _NW)


def _deg_body(dst_hbm, dep_hbm, out_hbm, idx_v, hist_v):
    del dep_hbm
    c = lax.axis_index("c")
    s = lax.axis_index("s")
    wid = s * _NC + c
    pltpu.sync_copy(dst_hbm.at[pl.ds(wid * _EW, _EW)], idx_v)
    zero16 = jnp.zeros((_L,), jnp.float32)
    one16 = jnp.ones((_L,), jnp.float32)

    def zbody(i, carry):
        hist_v[pl.ds(i * _L, _L)] = zero16
        return carry

    lax.fori_loop(0, _N // _L, zbody, 0)

    def hbody(i, carry):
        idx = idx_v[pl.ds(i * _L, _L)]
        plsc.addupdate_scatter(hist_v, [idx], one16)
        return carry

    lax.fori_loop(0, _EW // _L, hbody, 0)
    pltpu.sync_copy(hist_v, out_hbm.at[wid])


@functools.cache
def _deg_call():
    return pl.kernel(
        _deg_body,
        out_type=jax.ShapeDtypeStruct((_NW, _N), jnp.float32),
        mesh=_sc_mesh(),
        scratch_types=[
            pltpu.VMEM((_EW,), jnp.int32),
            pltpu.VMEM((_N,), jnp.float32),
        ],
        compiler_params=pltpu.CompilerParams(needs_layout_passes=False),
    )


def _scat_body(tab_hbm, src_hbm, dst_hbm, zero_hbm, out_hbm,
               sidx_v, didx_v, row_v, acc_sh,
               gsem_a, gsem_b, ssem_a, ssem_b):
    c = lax.axis_index("c")
    s = lax.axis_index("s")
    wid = s * _NC + c
    pltpu.sync_copy(zero_hbm.at[pl.ds(s * _RPS, _RPS)],
                    acc_sh.at[pl.ds(s * _RPS, _RPS)])
    pltpu.sync_copy(src_hbm.at[pl.ds(wid * _NCHUNK, _NCHUNK)], sidx_v)
    pltpu.sync_copy(dst_hbm.at[pl.ds(wid * _NCHUNK, _NCHUNK)], didx_v)
    plsc.subcore_barrier()

    row_a = row_v.at[0]
    row_b = row_v.at[1]

    def _gather(j, row, sem):
        pltpu.async_copy(tab_hbm.at[sidx_v.at[j]], row, sem)

    def _gwait(j, row, sem):
        pltpu.make_async_copy(tab_hbm.at[sidx_v.at[j]], row, sem).wait()

    def _scat(j, row, sem):
        pltpu.async_copy(row, acc_sh.at[didx_v.at[j]], sem, add=True)

    def _swait(j, row, sem):
        pltpu.make_async_copy(row, acc_sh.at[didx_v.at[j]], sem).wait()

    _gather(0, row_a, gsem_a)

    def body(i, carry):
        ja = 2 * i
        jb = 2 * i + 1
        _gwait(ja, row_a, gsem_a)
        _scat(ja, row_a, ssem_a)

        @pl.when(i > 0)
        def _():
            _swait(jb - 2, row_b, ssem_b)

        _gather(jb, row_b, gsem_b)
        _gwait(jb, row_b, gsem_b)
        _scat(jb, row_b, ssem_b)
        _swait(ja, row_a, ssem_a)

        @pl.when(i < _NCHUNK // 2 - 1)
        def _():
            _gather(ja + 2, row_a, gsem_a)

        return carry

    lax.fori_loop(0, _NCHUNK // 2, body, 0)
    _swait(_NCHUNK - 1, row_b, ssem_b)
    plsc.subcore_barrier()
    pltpu.sync_copy(acc_sh.at[pl.ds(s * _RPS, _RPS)],
                    out_hbm.at[c, pl.ds(s * _RPS, _RPS)])


@functools.cache
def _scat_call():
    return pl.kernel(
        _scat_body,
        out_type=jax.ShapeDtypeStruct((_NC, _N, _HID), jnp.float32),
        mesh=_sc_mesh(),
        scratch_types=[
            pltpu.VMEM((_NCHUNK, _CH), jnp.int32),
            pltpu.VMEM((_NCHUNK, _CH), jnp.int32),
            pltpu.VMEM((2, _CH, _HID), jnp.float32),
            pltpu.VMEM_SHARED((_N, _HID), jnp.float32),
            pltpu.SemaphoreType.DMA,
            pltpu.SemaphoreType.DMA,
            pltpu.SemaphoreType.DMA,
            pltpu.SemaphoreType.DMA,
        ],
        compiler_params=pltpu.CompilerParams(needs_layout_passes=False,
                                             use_tc_tiling_on_sc=False),
    )


def _feat_body(x_hbm, tri_hbm, out_hbm, xrow_v, tri_v, feat_v):
    c = lax.axis_index("c")
    s = lax.axis_index("s")
    wid = s * _NC + c
    pltpu.sync_copy(tri_hbm, tri_v)
    for k in range(_GPW):
        g = wid + _NW * k

        @pl.when(g < _NG)
        def _():
            pltpu.sync_copy(x_hbm.at[g], xrow_v)

            def gbody(i, carry):
                idx = tri_v[pl.ds(i * _L, _L)]
                feat_v[pl.ds(i * _L, _L)] = plsc.load_gather(xrow_v, [idx])
                return carry

            lax.fori_loop(0, _TRI // _L, gbody, 0)
            pltpu.sync_copy(feat_v, out_hbm.at[g])


@functools.cache
def _feat_call():
    return pl.kernel(
        _feat_body,
        out_type=jax.ShapeDtypeStruct((_NG, _TRI), jnp.float32),
        mesh=_sc_mesh(),
        scratch_types=[
            pltpu.VMEM((_F * _F,), jnp.float32),
            pltpu.VMEM((_TRI,), jnp.int32),
            pltpu.VMEM((_TRI,), jnp.float32),
        ],
        compiler_params=pltpu.CompilerParams(needs_layout_passes=False,
                                             use_tc_tiling_on_sc=False),
    )


def _ga_body(x_ref, w0_ref, d32_ref, g0_ref, dinv_ref):
    deg = jnp.sum(d32_ref[...], axis=1, keepdims=True) + 1.0
    dinv = lax.rsqrt(jnp.maximum(deg, 1.0))
    hw = jnp.dot(x_ref[...], w0_ref[...], preferred_element_type=jnp.float32)
    g0_ref[...] = hw * dinv
    dinv_ref[...] = dinv


def _ga(x, w0, d32t):
    return pl.pallas_call(
        _ga_body,
        out_shape=(
            jax.ShapeDtypeStruct((_N, _HID), jnp.float32),
            jax.ShapeDtypeStruct((_N, 1), jnp.float32),
        ),
    )(x, w0, d32t)


def _c_body(m_ref, g0_ref, dinv_ref, b0_ref, w1_ref, h1_ref, g1_ref):
    dinv = dinv_ref[...]
    h1 = jnp.tanh((m_ref[0] + m_ref[1] + g0_ref[...]) * dinv + b0_ref[...])
    h1_ref[...] = h1
    g1_ref[...] = jnp.dot(h1, w1_ref[...],
                          preferred_element_type=jnp.float32) * dinv


def _c(m0, g0, dinv, b0, w1):
    return pl.pallas_call(
        _c_body,
        out_shape=(
            jax.ShapeDtypeStruct((_N, _HID), jnp.float32),
            jax.ShapeDtypeStruct((_N, _HID), jnp.float32),
        ),
    )(m0, g0, dinv, b0, w1)


def _d1_body(m_ref, g1_ref, dinv_ref, b1_ref, h1_ref, bg_ref, bb_ref, out_ref):
    dinv = dinv_ref[...]
    h2 = jnp.tanh((m_ref[0] + m_ref[1] + g1_ref[...]) * dinv + b1_ref[...])
    hcat = jnp.concatenate([h1_ref[...], h2], axis=1)
    rr = lax.broadcasted_iota(jnp.int32, (_NG, _N), 0)
    cc = lax.broadcasted_iota(jnp.int32, (_NG, _N), 1)
    sel = jnp.where((cc // _F) == rr, jnp.float32(1.0 / _F), jnp.float32(0.0))
    m = jnp.dot(sel, hcat, preferred_element_type=jnp.float32)
    mu = jnp.mean(m, axis=0, keepdims=True)
    var = jnp.mean((m - mu) ** 2, axis=0, keepdims=True)
    out_ref[...] = (m - mu) * lax.rsqrt(var + 1e-5) * bg_ref[...] + bb_ref[...]


def _d1(m1, g1, dinv, b1, h1, bg, bb):
    return pl.pallas_call(
        _d1_body,
        out_shape=jax.ShapeDtypeStruct((_NG, _F), jnp.float32),
    )(m1, g1, dinv, b1, h1, bg, bb)


def _bn_relu(z, g, b):
    mu = jnp.mean(z, axis=0, keepdims=True)
    var = jnp.mean((z - mu) ** 2, axis=0, keepdims=True)
    return jnp.maximum((z - mu) * lax.rsqrt(var + 1e-5) * g + b,
                       jnp.float32(0.0))


def _d2_body(feat_ref, gf_ref, bf_ref, hbn_ref, wf_ref, wh_ref,
             b0_ref, g0_ref, bb0_ref, w1_ref, b1_ref, g1_ref, bb1_ref,
             w2_ref, b2_ref, g2_ref, bb2_ref, w3_ref, b3_ref, out_ref):
    f = feat_ref[...]
    mu = jnp.mean(f, axis=0, keepdims=True)
    var = jnp.mean((f - mu) ** 2, axis=0, keepdims=True)
    fbn = (f - mu) * lax.rsqrt(var + 1e-5) * gf_ref[...] + bf_ref[...]
    z = (jnp.dot(fbn, wf_ref[...], preferred_element_type=jnp.float32)
         + jnp.dot(hbn_ref[...], wh_ref[...],
                   preferred_element_type=jnp.float32)
         + b0_ref[...])
    z = _bn_relu(z, g0_ref[...], bb0_ref[...])
    z = _bn_relu(jnp.dot(z, w1_ref[...], preferred_element_type=jnp.float32)
                 + b1_ref[...], g1_ref[...], bb1_ref[...])
    z = _bn_relu(jnp.dot(z, w2_ref[...], preferred_element_type=jnp.float32)
                 + b2_ref[...], g2_ref[...], bb2_ref[...])
    out_ref[...] = (jnp.dot(z, w3_ref[...], preferred_element_type=jnp.float32)
                    + b3_ref[...])


def _d2(feat, gf, bf, hbn, wf, wh, p):
    return pl.pallas_call(
        _d2_body,
        out_shape=jax.ShapeDtypeStruct((_NG, 2), jnp.float32),
    )(feat, gf, bf, hbn, wf, wh,
      p["mlp0_b"].reshape(1, -1), p["mbn0_g"].reshape(1, -1),
      p["mbn0_b"].reshape(1, -1),
      p["mlp1_W"], p["mlp1_b"].reshape(1, -1), p["mbn1_g"].reshape(1, -1),
      p["mbn1_b"].reshape(1, -1),
      p["mlp2_W"], p["mlp2_b"].reshape(1, -1), p["mbn2_g"].reshape(1, -1),
      p["mbn2_b"].reshape(1, -1),
      p["mlp3_W"], p["mlp3_b"].reshape(1, -1))


def kernel(x, edge_index, batch, params):
    del batch
    p = params
    src_flat = edge_index[0]
    dst_flat = edge_index[1]
    src_r = src_flat.reshape(_NW * _NCHUNK, _CH)
    dst_r = dst_flat.reshape(_NW * _NCHUNK, _CH)
    zeros_tab = jnp.zeros((_N, _HID), jnp.float32)

    feat = _feat_call()(x.reshape(_NG, _F * _F), jnp.asarray(_TRIU_FLAT))
    d32 = _deg_call()(dst_flat, feat)
    g0, dinv = _ga(x, p["conv0_W"], d32.T)
    m0 = _scat_call()(g0, src_r, dst_r, zeros_tab)
    h1, g1 = _c(m0, g0, dinv, p["conv0_b"].reshape(1, -1), p["conv1_W"])
    m1 = _scat_call()(g1, src_r, dst_r, zeros_tab)
    hbn = _d1(m1, g1, dinv, p["conv1_b"].reshape(1, -1), h1,
              p["bnh_g"].reshape(1, -1), p["bnh_b"].reshape(1, -1))

    wf = p["mlp0_W"][:_TRI]
    wh = p["mlp0_W"][_TRI:]
    return _d2(feat, p["bn_g"].reshape(1, -1), p["bn_b"].reshape(1, -1),
               hbn, wf, wh, p)


# speedup vs baseline: 39.1977x; 1.3542x over previous
import functools

import numpy as np
import jax
import jax.numpy as jnp
from jax import lax
from jax.experimental import pallas as pl
from jax.experimental.pallas import tpu as pltpu
from jax.experimental.pallas import tpu_sc as plsc

_NG = 100
_F = 128
_N = _NG * _F
_E = 409600
_HID = 64
_NC, _NS, _L = 2, 16, 16
_NW = _NC * _NS
_EW = _E // _NW
_CH = 128
_NCHUNK = _EW // _CH
_RPS = _N // _NS

@functools.cache
def _sc_mesh():
    return plsc.VectorSubcoreMesh(core_axis_name="c", subcore_axis_name="s",
                                  num_cores=_NC, num_subcores=_NS)

_IU = np.triu_indices(_F, 1)
_TRIU_FLAT = (_IU[0] * _F + _IU[1]).astype(np.int32)
_TRI = _TRIU_FLAT.size
_GPW = -(-_NG // _NW)


def _deg_body(dst_hbm, out_hbm, idx_v, hist_v):
    c = lax.axis_index("c")
    s = lax.axis_index("s")
    wid = s * _NC + c
    pltpu.sync_copy(dst_hbm.at[pl.ds(wid * _EW, _EW)], idx_v)
    zero16 = jnp.zeros((_L,), jnp.float32)
    one16 = jnp.ones((_L,), jnp.float32)

    def zbody(i, carry):
        hist_v[pl.ds(i * _L, _L)] = zero16
        return carry

    lax.fori_loop(0, _N // _L, zbody, 0)

    def hbody(i, carry):
        idx = idx_v[pl.ds(i * _L, _L)]
        plsc.addupdate_scatter(hist_v, [idx], one16)
        return carry

    lax.fori_loop(0, _EW // _L, hbody, 0)
    pltpu.sync_copy(hist_v, out_hbm.at[wid])


@functools.cache
def _deg_call():
    return pl.kernel(
        _deg_body,
        out_type=jax.ShapeDtypeStruct((_NW, _N), jnp.float32),
        mesh=_sc_mesh(),
        scratch_types=[
            pltpu.VMEM((_EW,), jnp.int32),
            pltpu.VMEM((_N,), jnp.float32),
        ],
        compiler_params=pltpu.CompilerParams(needs_layout_passes=False),
    )


def _scat_body(tab_hbm, src_hbm, dst_hbm, zero_hbm, out_hbm,
               sidx_v, didx_v, row_v, acc_sh,
               gsems, ssems):
    c = lax.axis_index("c")
    s = lax.axis_index("s")
    wid = s * _NC + c
    pltpu.sync_copy(zero_hbm.at[pl.ds(s * _RPS, _RPS)],
                    acc_sh.at[pl.ds(s * _RPS, _RPS)])
    pltpu.sync_copy(src_hbm.at[pl.ds(wid * _NCHUNK, _NCHUNK)], sidx_v)
    pltpu.sync_copy(dst_hbm.at[pl.ds(wid * _NCHUNK, _NCHUNK)], didx_v)
    plsc.subcore_barrier()

    def _gather(j, b):
        pltpu.async_copy(tab_hbm.at[sidx_v.at[j]], row_v.at[b], gsems.at[b])

    def _gwait(j, b):
        pltpu.make_async_copy(tab_hbm.at[sidx_v.at[j]], row_v.at[b],
                              gsems.at[b]).wait()

    def _scat(j, b):
        pltpu.async_copy(row_v.at[b], acc_sh.at[didx_v.at[j]], ssems.at[b],
                         add=True)

    def _swait(j, b):
        pltpu.make_async_copy(row_v.at[b], acc_sh.at[didx_v.at[j]],
                              ssems.at[b]).wait()

    def step(j, b):
        @pl.when(j >= 4)
        def _():
            _swait(j - 4, b)

        _gather(j, b)

        @pl.when(j >= 2)
        def _():
            _gwait(j - 2, (b + 2) % 4)
            _scat(j - 2, (b + 2) % 4)

    def body(i, carry):
        for b in range(4):
            step(4 * i + b, b)
        return carry

    lax.fori_loop(0, _NCHUNK // 4, body, 0)
    for j in (_NCHUNK - 2, _NCHUNK - 1):
        _gwait(j, j % 4)
        _scat(j, j % 4)
    for j in range(_NCHUNK - 4, _NCHUNK):
        _swait(j, j % 4)
    plsc.subcore_barrier()
    pltpu.sync_copy(acc_sh.at[pl.ds(s * _RPS, _RPS)],
                    out_hbm.at[c, pl.ds(s * _RPS, _RPS)])


@functools.cache
def _scat_call():
    return pl.kernel(
        _scat_body,
        out_type=jax.ShapeDtypeStruct((_NC, _N, _HID), jnp.float32),
        mesh=_sc_mesh(),
        scratch_types=[
            pltpu.VMEM((_NCHUNK, _CH), jnp.int32),
            pltpu.VMEM((_NCHUNK, _CH), jnp.int32),
            pltpu.VMEM((4, _CH, _HID), jnp.float32),
            pltpu.VMEM_SHARED((_N, _HID), jnp.float32),
            pltpu.SemaphoreType.DMA((4,)),
            pltpu.SemaphoreType.DMA((4,)),
        ],
        compiler_params=pltpu.CompilerParams(needs_layout_passes=False,
                                             use_tc_tiling_on_sc=False),
    )


def _feat_body(x_hbm, tri_hbm, dep_hbm, out_hbm, xrow_v, tri_v, feat_v):
    del dep_hbm
    c = lax.axis_index("c")
    s = lax.axis_index("s")
    wid = s * _NC + c
    pltpu.sync_copy(tri_hbm, tri_v)
    for k in range(_GPW):
        g = wid + _NW * k

        @pl.when(g < _NG)
        def _():
            pltpu.sync_copy(x_hbm.at[g], xrow_v)

            def gbody(i, carry):
                idx = tri_v[pl.ds(i * _L, _L)]
                feat_v[pl.ds(i * _L, _L)] = plsc.load_gather(xrow_v, [idx])
                return carry

            lax.fori_loop(0, _TRI // _L, gbody, 0)
            pltpu.sync_copy(feat_v, out_hbm.at[g])


@functools.cache
def _feat_call():
    return pl.kernel(
        _feat_body,
        out_type=jax.ShapeDtypeStruct((_NG, _TRI), jnp.float32),
        mesh=_sc_mesh(),
        scratch_types=[
            pltpu.VMEM((_F * _F,), jnp.float32),
            pltpu.VMEM((_TRI,), jnp.int32),
            pltpu.VMEM((_TRI,), jnp.float32),
        ],
        compiler_params=pltpu.CompilerParams(needs_layout_passes=False,
                                             use_tc_tiling_on_sc=False),
    )


def _ga_body(x_ref, w0_ref, d32_ref, g0_ref, dinv_ref):
    deg = jnp.sum(d32_ref[...], axis=1, keepdims=True) + 1.0
    dinv = lax.rsqrt(jnp.maximum(deg, 1.0))
    hw = jnp.dot(x_ref[...], w0_ref[...], preferred_element_type=jnp.float32)
    g0_ref[...] = hw * dinv
    dinv_ref[...] = dinv


def _ga(x, w0, d32t):
    return pl.pallas_call(
        _ga_body,
        out_shape=(
            jax.ShapeDtypeStruct((_N, _HID), jnp.float32),
            jax.ShapeDtypeStruct((_N, 1), jnp.float32),
        ),
    )(x, w0, d32t)


def _c_body(m_ref, g0_ref, dinv_ref, b0_ref, w1_ref, h1_ref, g1_ref):
    dinv = dinv_ref[...]
    h1 = jnp.tanh((m_ref[0] + m_ref[1] + g0_ref[...]) * dinv + b0_ref[...])
    h1_ref[...] = h1
    g1_ref[...] = jnp.dot(h1, w1_ref[...],
                          preferred_element_type=jnp.float32) * dinv


def _c(m0, g0, dinv, b0, w1):
    return pl.pallas_call(
        _c_body,
        out_shape=(
            jax.ShapeDtypeStruct((_N, _HID), jnp.float32),
            jax.ShapeDtypeStruct((_N, _HID), jnp.float32),
        ),
    )(m0, g0, dinv, b0, w1)


def _d1_body(m_ref, g1_ref, dinv_ref, b1_ref, h1_ref, bg_ref, bb_ref, out_ref):
    dinv = dinv_ref[...]
    h2 = jnp.tanh((m_ref[0] + m_ref[1] + g1_ref[...]) * dinv + b1_ref[...])
    hcat = jnp.concatenate([h1_ref[...], h2], axis=1)
    rr = lax.broadcasted_iota(jnp.int32, (_NG, _N), 0)
    cc = lax.broadcasted_iota(jnp.int32, (_NG, _N), 1)
    sel = jnp.where((cc // _F) == rr, jnp.float32(1.0 / _F), jnp.float32(0.0))
    m = jnp.dot(sel, hcat, preferred_element_type=jnp.float32)
    mu = jnp.mean(m, axis=0, keepdims=True)
    var = jnp.mean((m - mu) ** 2, axis=0, keepdims=True)
    out_ref[...] = (m - mu) * lax.rsqrt(var + 1e-5) * bg_ref[...] + bb_ref[...]


def _d1(m1, g1, dinv, b1, h1, bg, bb):
    return pl.pallas_call(
        _d1_body,
        out_shape=jax.ShapeDtypeStruct((_NG, _F), jnp.float32),
    )(m1, g1, dinv, b1, h1, bg, bb)


def _bn_relu(z, g, b):
    mu = jnp.mean(z, axis=0, keepdims=True)
    var = jnp.mean((z - mu) ** 2, axis=0, keepdims=True)
    return jnp.maximum((z - mu) * lax.rsqrt(var + 1e-5) * g + b,
                       jnp.float32(0.0))


def _d2_body(feat_ref, gf_ref, bf_ref, hbn_ref, wf_ref, wh_ref,
             b0_ref, g0_ref, bb0_ref, w1_ref, b1_ref, g1_ref, bb1_ref,
             w2_ref, b2_ref, g2_ref, bb2_ref, w3_ref, b3_ref, out_ref):
    f = feat_ref[...]
    mu = jnp.mean(f, axis=0, keepdims=True)
    var = jnp.mean((f - mu) ** 2, axis=0, keepdims=True)
    fbn = (f - mu) * lax.rsqrt(var + 1e-5) * gf_ref[...] + bf_ref[...]
    z = (jnp.dot(fbn, wf_ref[...], preferred_element_type=jnp.float32)
         + jnp.dot(hbn_ref[...], wh_ref[...],
                   preferred_element_type=jnp.float32)
         + b0_ref[...])
    z = _bn_relu(z, g0_ref[...], bb0_ref[...])
    z = _bn_relu(jnp.dot(z, w1_ref[...], preferred_element_type=jnp.float32)
                 + b1_ref[...], g1_ref[...], bb1_ref[...])
    z = _bn_relu(jnp.dot(z, w2_ref[...], preferred_element_type=jnp.float32)
                 + b2_ref[...], g2_ref[...], bb2_ref[...])
    out_ref[...] = (jnp.dot(z, w3_ref[...], preferred_element_type=jnp.float32)
                    + b3_ref[...])


def _d2(feat, gf, bf, hbn, wf, wh, p):
    return pl.pallas_call(
        _d2_body,
        out_shape=jax.ShapeDtypeStruct((_NG, 2), jnp.float32),
    )(feat, gf, bf, hbn, wf, wh,
      p["mlp0_b"].reshape(1, -1), p["mbn0_g"].reshape(1, -1),
      p["mbn0_b"].reshape(1, -1),
      p["mlp1_W"], p["mlp1_b"].reshape(1, -1), p["mbn1_g"].reshape(1, -1),
      p["mbn1_b"].reshape(1, -1),
      p["mlp2_W"], p["mlp2_b"].reshape(1, -1), p["mbn2_g"].reshape(1, -1),
      p["mbn2_b"].reshape(1, -1),
      p["mlp3_W"], p["mlp3_b"].reshape(1, -1))


def kernel(x, edge_index, batch, params):
    del batch
    p = params
    src_flat = edge_index[0]
    dst_flat = edge_index[1]
    src_r = src_flat.reshape(_NW * _NCHUNK, _CH)
    dst_r = dst_flat.reshape(_NW * _NCHUNK, _CH)
    zeros_tab = jnp.zeros((_N, _HID), jnp.float32)

    d32 = _deg_call()(dst_flat)
    g0, dinv = _ga(x, p["conv0_W"], d32.T)
    m0 = _scat_call()(g0, src_r, dst_r, zeros_tab)
    h1, g1 = _c(m0, g0, dinv, p["conv0_b"].reshape(1, -1), p["conv1_W"])
    m1 = _scat_call()(g1, src_r, dst_r, zeros_tab)
    feat = _feat_call()(x.reshape(_NG, _F * _F), jnp.asarray(_TRIU_FLAT), m1)
    hbn = _d1(m1, g1, dinv, p["conv1_b"].reshape(1, -1), h1,
              p["bnh_g"].reshape(1, -1), p["bnh_b"].reshape(1, -1))

    wf = p["mlp0_W"][:_TRI]
    wh = p["mlp0_W"][_TRI:]
    return _d2(feat, p["bn_g"].reshape(1, -1), p["bn_b"].reshape(1, -1),
               hbn, wf, wh, p)


# speedup vs baseline: 40.0782x; 1.0225x over previous
import functools

import numpy as np
import jax
import jax.numpy as jnp
from jax import lax
from jax.experimental import pallas as pl
from jax.experimental.pallas import tpu as pltpu
from jax.experimental.pallas import tpu_sc as plsc

_NG = 100
_F = 128
_N = _NG * _F
_E = 409600
_HID = 64
_NC, _NS, _L = 2, 16, 16
_NW = _NC * _NS
_EW = _E // _NW
_CH = 128
_NCHUNK = _EW // _CH
_RPS = _N // _NS
_NBUF = 5

@functools.cache
def _sc_mesh():
    return plsc.VectorSubcoreMesh(core_axis_name="c", subcore_axis_name="s",
                                  num_cores=_NC, num_subcores=_NS)

_IU = np.triu_indices(_F, 1)
_TRIU_FLAT = (_IU[0] * _F + _IU[1]).astype(np.int32)
_TRI = _TRIU_FLAT.size
_GPW = -(-_NG // _NW)


def _deg_body(dst_hbm, out_hbm, idx_v, hist_v):
    c = lax.axis_index("c")
    s = lax.axis_index("s")
    wid = s * _NC + c
    pltpu.sync_copy(dst_hbm.at[pl.ds(wid * _EW, _EW)], idx_v)
    zero16 = jnp.zeros((_L,), jnp.float32)
    one16 = jnp.ones((_L,), jnp.float32)

    def zbody(i, carry):
        hist_v[pl.ds(i * _L, _L)] = zero16
        return carry

    lax.fori_loop(0, _N // _L, zbody, 0)

    def hbody(i, carry):
        idx = idx_v[pl.ds(i * _L, _L)]
        plsc.addupdate_scatter(hist_v, [idx], one16)
        return carry

    lax.fori_loop(0, _EW // _L, hbody, 0)
    pltpu.sync_copy(hist_v, out_hbm.at[wid])


@functools.cache
def _deg_call():
    return pl.kernel(
        _deg_body,
        out_type=jax.ShapeDtypeStruct((_NW, _N), jnp.float32),
        mesh=_sc_mesh(),
        scratch_types=[
            pltpu.VMEM((_EW,), jnp.int32),
            pltpu.VMEM((_N,), jnp.float32),
        ],
        compiler_params=pltpu.CompilerParams(needs_layout_passes=False),
    )


def _scat_body(tab_hbm, src_hbm, dst_hbm, zero_hbm, out_hbm,
               sidx_v, didx_v, row_v, acc_sh,
               gsems, ssems):
    c = lax.axis_index("c")
    s = lax.axis_index("s")
    wid = s * _NC + c
    pltpu.sync_copy(zero_hbm.at[pl.ds(s * _RPS, _RPS)],
                    acc_sh.at[pl.ds(s * _RPS, _RPS)])
    pltpu.sync_copy(src_hbm.at[pl.ds(wid * _NCHUNK, _NCHUNK)], sidx_v)
    pltpu.sync_copy(dst_hbm.at[pl.ds(wid * _NCHUNK, _NCHUNK)], didx_v)
    plsc.subcore_barrier()

    def _gather(j, b):
        pltpu.async_copy(tab_hbm.at[sidx_v.at[j]], row_v.at[b], gsems.at[b])

    def _gwait(j, b):
        pltpu.make_async_copy(tab_hbm.at[sidx_v.at[j]], row_v.at[b],
                              gsems.at[b]).wait()

    def _scat(j, b):
        pltpu.async_copy(row_v.at[b], acc_sh.at[didx_v.at[j]], ssems.at[b],
                         add=True)

    def _swait(j, b):
        pltpu.make_async_copy(row_v.at[b], acc_sh.at[didx_v.at[j]],
                              ssems.at[b]).wait()

    half = _NBUF // 2

    def step(j, b):
        @pl.when(j >= _NBUF)
        def _():
            _swait(j - _NBUF, b)

        _gather(j, b)

        @pl.when(j >= half)
        def _():
            _gwait(j - half, (b - half) % _NBUF)
            _scat(j - half, (b - half) % _NBUF)

    def body(i, carry):
        for b in range(_NBUF):
            step(_NBUF * i + b, b)
        return carry

    lax.fori_loop(0, _NCHUNK // _NBUF, body, 0)
    for j in range(_NCHUNK - half, _NCHUNK):
        _gwait(j, j % _NBUF)
        _scat(j, j % _NBUF)
    for j in range(_NCHUNK - _NBUF, _NCHUNK):
        _swait(j, j % _NBUF)
    plsc.subcore_barrier()
    pltpu.sync_copy(acc_sh.at[pl.ds(s * _RPS, _RPS)],
                    out_hbm.at[c, pl.ds(s * _RPS, _RPS)])


@functools.cache
def _scat_call():
    return pl.kernel(
        _scat_body,
        out_type=jax.ShapeDtypeStruct((_NC, _N, _HID), jnp.float32),
        mesh=_sc_mesh(),
        scratch_types=[
            pltpu.VMEM((_NCHUNK, _CH), jnp.int32),
            pltpu.VMEM((_NCHUNK, _CH), jnp.int32),
            pltpu.VMEM((_NBUF, _CH, _HID), jnp.float32),
            pltpu.VMEM_SHARED((_N, _HID), jnp.float32),
            pltpu.SemaphoreType.DMA((_NBUF,)),
            pltpu.SemaphoreType.DMA((_NBUF,)),
        ],
        compiler_params=pltpu.CompilerParams(needs_layout_passes=False,
                                             use_tc_tiling_on_sc=False),
    )


def _feat_body(x_hbm, tri_hbm, dep_hbm, out_hbm, xrow_v, tri_v, feat_v):
    del dep_hbm
    c = lax.axis_index("c")
    s = lax.axis_index("s")
    wid = s * _NC + c
    pltpu.sync_copy(tri_hbm, tri_v)
    for k in range(_GPW):
        g = wid + _NW * k

        @pl.when(g < _NG)
        def _():
            pltpu.sync_copy(x_hbm.at[g], xrow_v)

            def gbody(i, carry):
                idx = tri_v[pl.ds(i * _L, _L)]
                feat_v[pl.ds(i * _L, _L)] = plsc.load_gather(xrow_v, [idx])
                return carry

            lax.fori_loop(0, _TRI // _L, gbody, 0)
            pltpu.sync_copy(feat_v, out_hbm.at[g])


@functools.cache
def _feat_call():
    return pl.kernel(
        _feat_body,
        out_type=jax.ShapeDtypeStruct((_NG, _TRI), jnp.float32),
        mesh=_sc_mesh(),
        scratch_types=[
            pltpu.VMEM((_F * _F,), jnp.float32),
            pltpu.VMEM((_TRI,), jnp.int32),
            pltpu.VMEM((_TRI,), jnp.float32),
        ],
        compiler_params=pltpu.CompilerParams(needs_layout_passes=False,
                                             use_tc_tiling_on_sc=False),
    )


def _ga_body(x_ref, w0_ref, d32_ref, g0_ref, dinv_ref):
    deg = jnp.sum(d32_ref[...], axis=1, keepdims=True) + 1.0
    dinv = lax.rsqrt(jnp.maximum(deg, 1.0))
    hw = jnp.dot(x_ref[...], w0_ref[...], preferred_element_type=jnp.float32)
    g0_ref[...] = hw * dinv
    dinv_ref[...] = dinv


def _ga(x, w0, d32t):
    return pl.pallas_call(
        _ga_body,
        out_shape=(
            jax.ShapeDtypeStruct((_N, _HID), jnp.float32),
            jax.ShapeDtypeStruct((_N, 1), jnp.float32),
        ),
    )(x, w0, d32t)


def _c_body(m_ref, g0_ref, dinv_ref, b0_ref, w1_ref, h1_ref, g1_ref):
    dinv = dinv_ref[...]
    h1 = jnp.tanh((m_ref[0] + m_ref[1] + g0_ref[...]) * dinv + b0_ref[...])
    h1_ref[...] = h1
    g1_ref[...] = jnp.dot(h1, w1_ref[...],
                          preferred_element_type=jnp.float32) * dinv


def _c(m0, g0, dinv, b0, w1):
    return pl.pallas_call(
        _c_body,
        out_shape=(
            jax.ShapeDtypeStruct((_N, _HID), jnp.float32),
            jax.ShapeDtypeStruct((_N, _HID), jnp.float32),
        ),
    )(m0, g0, dinv, b0, w1)


def _d1_body(m_ref, g1_ref, dinv_ref, b1_ref, h1_ref, bg_ref, bb_ref, out_ref):
    dinv = dinv_ref[...]
    h2 = jnp.tanh((m_ref[0] + m_ref[1] + g1_ref[...]) * dinv + b1_ref[...])
    hcat = jnp.concatenate([h1_ref[...], h2], axis=1)
    rr = lax.broadcasted_iota(jnp.int32, (_NG, _N), 0)
    cc = lax.broadcasted_iota(jnp.int32, (_NG, _N), 1)
    sel = jnp.where((cc // _F) == rr, jnp.float32(1.0 / _F), jnp.float32(0.0))
    m = jnp.dot(sel, hcat, preferred_element_type=jnp.float32)
    mu = jnp.mean(m, axis=0, keepdims=True)
    var = jnp.mean((m - mu) ** 2, axis=0, keepdims=True)
    out_ref[...] = (m - mu) * lax.rsqrt(var + 1e-5) * bg_ref[...] + bb_ref[...]


def _d1(m1, g1, dinv, b1, h1, bg, bb):
    return pl.pallas_call(
        _d1_body,
        out_shape=jax.ShapeDtypeStruct((_NG, _F), jnp.float32),
    )(m1, g1, dinv, b1, h1, bg, bb)


def _bn_relu(z, g, b):
    mu = jnp.mean(z, axis=0, keepdims=True)
    var = jnp.mean((z - mu) ** 2, axis=0, keepdims=True)
    return jnp.maximum((z - mu) * lax.rsqrt(var + 1e-5) * g + b,
                       jnp.float32(0.0))


def _d2_body(feat_ref, gf_ref, bf_ref, hbn_ref, wf_ref, wh_ref,
             b0_ref, g0_ref, bb0_ref, w1_ref, b1_ref, g1_ref, bb1_ref,
             w2_ref, b2_ref, g2_ref, bb2_ref, w3_ref, b3_ref, out_ref):
    f = feat_ref[...]
    mu = jnp.mean(f, axis=0, keepdims=True)
    var = jnp.mean((f - mu) ** 2, axis=0, keepdims=True)
    fbn = (f - mu) * lax.rsqrt(var + 1e-5) * gf_ref[...] + bf_ref[...]
    z = (jnp.dot(fbn, wf_ref[...], preferred_element_type=jnp.float32)
         + jnp.dot(hbn_ref[...], wh_ref[...],
                   preferred_element_type=jnp.float32)
         + b0_ref[...])
    z = _bn_relu(z, g0_ref[...], bb0_ref[...])
    z = _bn_relu(jnp.dot(z, w1_ref[...], preferred_element_type=jnp.float32)
                 + b1_ref[...], g1_ref[...], bb1_ref[...])
    z = _bn_relu(jnp.dot(z, w2_ref[...], preferred_element_type=jnp.float32)
                 + b2_ref[...], g2_ref[...], bb2_ref[...])
    out_ref[...] = (jnp.dot(z, w3_ref[...], preferred_element_type=jnp.float32)
                    + b3_ref[...])


def _d2(feat, gf, bf, hbn, wf, wh, p):
    return pl.pallas_call(
        _d2_body,
        out_shape=jax.ShapeDtypeStruct((_NG, 2), jnp.float32),
    )(feat, gf, bf, hbn, wf, wh,
      p["mlp0_b"].reshape(1, -1), p["mbn0_g"].reshape(1, -1),
      p["mbn0_b"].reshape(1, -1),
      p["mlp1_W"], p["mlp1_b"].reshape(1, -1), p["mbn1_g"].reshape(1, -1),
      p["mbn1_b"].reshape(1, -1),
      p["mlp2_W"], p["mlp2_b"].reshape(1, -1), p["mbn2_g"].reshape(1, -1),
      p["mbn2_b"].reshape(1, -1),
      p["mlp3_W"], p["mlp3_b"].reshape(1, -1))


def kernel(x, edge_index, batch, params):
    del batch
    p = params
    src_flat = edge_index[0]
    dst_flat = edge_index[1]
    src_r = src_flat.reshape(_NW * _NCHUNK, _CH)
    dst_r = dst_flat.reshape(_NW * _NCHUNK, _CH)
    zeros_tab = jnp.zeros((_N, _HID), jnp.float32)

    d32 = _deg_call()(dst_flat)
    g0, dinv = _ga(x, p["conv0_W"], d32.T)
    m0 = _scat_call()(g0, src_r, dst_r, zeros_tab)
    h1, g1 = _c(m0, g0, dinv, p["conv0_b"].reshape(1, -1), p["conv1_W"])
    m1 = _scat_call()(g1, src_r, dst_r, zeros_tab)
    feat = _feat_call()(x.reshape(_NG, _F * _F), jnp.asarray(_TRIU_FLAT), m1)
    hbn = _d1(m1, g1, dinv, p["conv1_b"].reshape(1, -1), h1,
              p["bnh_g"].reshape(1, -1), p["bnh_b"].reshape(1, -1))

    wf = p["mlp0_W"][:_TRI]
    wh = p["mlp0_W"][_TRI:]
    return _d2(feat, p["bn_g"].reshape(1, -1), p["bn_b"].reshape(1, -1),
               hbn, wf, wh, p)


# speedup vs baseline: 44.4874x; 1.1100x over previous
import functools

import numpy as np
import jax
import jax.numpy as jnp
from jax import lax
from jax.experimental import pallas as pl
from jax.experimental.pallas import tpu as pltpu
from jax.experimental.pallas import tpu_sc as plsc

_NG = 100
_F = 128
_N = _NG * _F
_E = 409600
_HID = 64
_NC, _NS, _L = 2, 16, 16
_NW = _NC * _NS
_EW = _E // _NW
_CH = 128
_NCHUNK = _EW // _CH
_RPS = _N // _NS
_NBUF = 5

@functools.cache
def _sc_mesh():
    return plsc.VectorSubcoreMesh(core_axis_name="c", subcore_axis_name="s",
                                  num_cores=_NC, num_subcores=_NS)

_IU = np.triu_indices(_F, 1)
_TRIU_FLAT = (_IU[0] * _F + _IU[1]).astype(np.int32)
_TRI = _TRIU_FLAT.size
_GPW = -(-_NG // _NW)


def _deg_body(dst_hbm, out_hbm, idx_v, hist_v):
    c = lax.axis_index("c")
    s = lax.axis_index("s")
    wid = s * _NC + c
    pltpu.sync_copy(dst_hbm.at[pl.ds(wid * _EW, _EW)], idx_v)
    zero16 = jnp.zeros((_L,), jnp.float32)
    one16 = jnp.ones((_L,), jnp.float32)

    def zbody(i, carry):
        hist_v[pl.ds(i * _L, _L)] = zero16
        return carry

    lax.fori_loop(0, _N // _L, zbody, 0)

    def hbody(i, carry):
        idx = idx_v[pl.ds(i * _L, _L)]
        plsc.addupdate_scatter(hist_v, [idx], one16)
        return carry

    lax.fori_loop(0, _EW // _L, hbody, 0)
    pltpu.sync_copy(hist_v, out_hbm.at[wid])


@functools.cache
def _deg_call():
    return pl.kernel(
        _deg_body,
        out_type=jax.ShapeDtypeStruct((_NW, _N), jnp.float32),
        mesh=_sc_mesh(),
        scratch_types=[
            pltpu.VMEM((_EW,), jnp.int32),
            pltpu.VMEM((_N,), jnp.float32),
        ],
        compiler_params=pltpu.CompilerParams(needs_layout_passes=False),
    )


def _scat_body(tab_hbm, src_hbm, dst_hbm, zero_hbm, out_hbm,
               sidx_v, didx_v, row_v, widx_v, acc_sh,
               gsems, ssems):
    c = lax.axis_index("c")
    s = lax.axis_index("s")
    wid = s * _NC + c
    pltpu.sync_copy(zero_hbm.at[pl.ds(s * _RPS, _RPS)],
                    acc_sh.at[pl.ds(s * _RPS, _RPS)])
    iota16 = lax.broadcasted_iota(jnp.int32, (_L,), 0)
    for t in range(8):
        for off in (0, 16, 32, 48, 64, 80, 84):
            vals = (iota16 + (s * _RPS + t * 100 + off)) * 2
            widx_v[t, pl.ds(off, _L)] = vals
    pltpu.sync_copy(src_hbm.at[pl.ds(wid * _NCHUNK, _NCHUNK)], sidx_v)
    pltpu.sync_copy(dst_hbm.at[pl.ds(wid * _NCHUNK, _NCHUNK)], didx_v)
    plsc.subcore_barrier()

    def _gather(j, b):
        pltpu.async_copy(tab_hbm.at[sidx_v.at[j]], row_v.at[b], gsems.at[b])

    def _gwait(j, b):
        pltpu.make_async_copy(tab_hbm.at[sidx_v.at[j]], row_v.at[b],
                              gsems.at[b]).wait()

    def _scat(j, b):
        pltpu.async_copy(row_v.at[b], acc_sh.at[didx_v.at[j]], ssems.at[b],
                         add=True)

    def _swait(j, b):
        pltpu.make_async_copy(row_v.at[b], acc_sh.at[didx_v.at[j]],
                              ssems.at[b]).wait()

    half = _NBUF // 2

    def step(j, b):
        @pl.when(j >= _NBUF)
        def _():
            _swait(j - _NBUF, b)

        _gather(j, b)

        @pl.when(j >= half)
        def _():
            _gwait(j - half, (b - half) % _NBUF)
            _scat(j - half, (b - half) % _NBUF)

    def body(i, carry):
        for b in range(_NBUF):
            step(_NBUF * i + b, b)
        return carry

    lax.fori_loop(0, _NCHUNK // _NBUF, body, 0)
    for j in range(_NCHUNK - half, _NCHUNK):
        _gwait(j, j % _NBUF)
        _scat(j, j % _NBUF)
    for j in range(_NCHUNK - _NBUF, _NCHUNK):
        _swait(j, j % _NBUF)
    plsc.subcore_barrier()
    for t in range(8):
        bounce = row_v.at[t % _NBUF, pl.ds(0, 100)]
        pltpu.sync_copy(acc_sh.at[pl.ds(s * _RPS + t * 100, 100)], bounce)
        pltpu.sync_copy(bounce, out_hbm.at[c].at[widx_v.at[t]])


@functools.cache
def _scat_call():
    return pl.kernel(
        _scat_body,
        out_type=jax.ShapeDtypeStruct((_NC, 2 * _N, _HID), jnp.float32),
        mesh=_sc_mesh(),
        scratch_types=[
            pltpu.VMEM((_NCHUNK, _CH), jnp.int32),
            pltpu.VMEM((_NCHUNK, _CH), jnp.int32),
            pltpu.VMEM((_NBUF, _CH, _HID), jnp.float32),
            pltpu.VMEM((8, 100), jnp.int32),
            pltpu.VMEM_SHARED((_N, _HID), jnp.float32),
            pltpu.SemaphoreType.DMA((_NBUF,)),
            pltpu.SemaphoreType.DMA((_NBUF,)),
        ],
        compiler_params=pltpu.CompilerParams(needs_layout_passes=False,
                                             use_tc_tiling_on_sc=False),
    )


def _feat_body(x_hbm, tri_hbm, dep_hbm, out_hbm, xrow_v, tri_v, feat_v):
    del dep_hbm
    c = lax.axis_index("c")
    s = lax.axis_index("s")
    wid = s * _NC + c
    pltpu.sync_copy(tri_hbm, tri_v)
    for k in range(_GPW):
        g = wid + _NW * k

        @pl.when(g < _NG)
        def _():
            pltpu.sync_copy(x_hbm.at[g], xrow_v)

            def gbody(i, carry):
                idx = tri_v[pl.ds(i * _L, _L)]
                feat_v[pl.ds(i * _L, _L)] = plsc.load_gather(xrow_v, [idx])
                return carry

            lax.fori_loop(0, _TRI // _L, gbody, 0)
            pltpu.sync_copy(feat_v, out_hbm.at[g])


@functools.cache
def _feat_call():
    return pl.kernel(
        _feat_body,
        out_type=jax.ShapeDtypeStruct((_NG, _TRI), jnp.float32),
        mesh=_sc_mesh(),
        scratch_types=[
            pltpu.VMEM((_F * _F,), jnp.float32),
            pltpu.VMEM((_TRI,), jnp.int32),
            pltpu.VMEM((_TRI,), jnp.float32),
        ],
        compiler_params=pltpu.CompilerParams(needs_layout_passes=False,
                                             use_tc_tiling_on_sc=False),
    )


def _ga_body(x_ref, w0_ref, d32_ref, g0_ref, dinv_ref):
    deg = jnp.sum(d32_ref[...], axis=1, keepdims=True) + 1.0
    dinv = lax.rsqrt(jnp.maximum(deg, 1.0))
    hw = jnp.dot(x_ref[...], w0_ref[...], preferred_element_type=jnp.float32)
    g0_ref[...] = jnp.concatenate(
        [hw * dinv, jnp.zeros((_N, _HID), jnp.float32)], axis=1)
    dinv_ref[...] = dinv


def _ga(x, w0, d32t):
    return pl.pallas_call(
        _ga_body,
        out_shape=(
            jax.ShapeDtypeStruct((_N, 2 * _HID), jnp.float32),
            jax.ShapeDtypeStruct((_N, 1), jnp.float32),
        ),
    )(x, w0, d32t)


def _c_body(m_ref, g0_ref, dinv_ref, b0_ref, w1_ref, h1_ref, g1_ref):
    dinv = dinv_ref[...]
    m = m_ref[0, :, :_HID] + m_ref[1, :, :_HID] + g0_ref[:, :_HID]
    h1 = jnp.tanh(m * dinv + b0_ref[...])
    h1_ref[...] = h1
    g1 = jnp.dot(h1, w1_ref[...], preferred_element_type=jnp.float32) * dinv
    g1_ref[...] = jnp.concatenate(
        [g1, jnp.zeros((_N, _HID), jnp.float32)], axis=1)


def _c(m0, g0, dinv, b0, w1):
    return pl.pallas_call(
        _c_body,
        out_shape=(
            jax.ShapeDtypeStruct((_N, _HID), jnp.float32),
            jax.ShapeDtypeStruct((_N, 2 * _HID), jnp.float32),
        ),
    )(m0, g0, dinv, b0, w1)


def _d1_body(m_ref, g1_ref, dinv_ref, b1_ref, h1_ref, bg_ref, bb_ref, out_ref):
    dinv = dinv_ref[...]
    m = m_ref[0, :, :_HID] + m_ref[1, :, :_HID] + g1_ref[:, :_HID]
    h2 = jnp.tanh(m * dinv + b1_ref[...])
    hcat = jnp.concatenate([h1_ref[...], h2], axis=1)
    rr = lax.broadcasted_iota(jnp.int32, (_NG, _N), 0)
    cc = lax.broadcasted_iota(jnp.int32, (_NG, _N), 1)
    sel = jnp.where((cc // _F) == rr, jnp.float32(1.0 / _F), jnp.float32(0.0))
    m = jnp.dot(sel, hcat, preferred_element_type=jnp.float32)
    mu = jnp.mean(m, axis=0, keepdims=True)
    var = jnp.mean((m - mu) ** 2, axis=0, keepdims=True)
    out_ref[...] = (m - mu) * lax.rsqrt(var + 1e-5) * bg_ref[...] + bb_ref[...]


def _d1(m1, g1, dinv, b1, h1, bg, bb):
    return pl.pallas_call(
        _d1_body,
        out_shape=jax.ShapeDtypeStruct((_NG, _F), jnp.float32),
    )(m1, g1, dinv, b1, h1, bg, bb)


def _bn_relu(z, g, b):
    mu = jnp.mean(z, axis=0, keepdims=True)
    var = jnp.mean((z - mu) ** 2, axis=0, keepdims=True)
    return jnp.maximum((z - mu) * lax.rsqrt(var + 1e-5) * g + b,
                       jnp.float32(0.0))


def _d2_body(feat_ref, gf_ref, bf_ref, hbn_ref, wf_ref, wh_ref,
             b0_ref, g0_ref, bb0_ref, w1_ref, b1_ref, g1_ref, bb1_ref,
             w2_ref, b2_ref, g2_ref, bb2_ref, w3_ref, b3_ref, out_ref):
    f = feat_ref[...]
    mu = jnp.mean(f, axis=0, keepdims=True)
    var = jnp.mean((f - mu) ** 2, axis=0, keepdims=True)
    fbn = (f - mu) * lax.rsqrt(var + 1e-5) * gf_ref[...] + bf_ref[...]
    z = (jnp.dot(fbn, wf_ref[...], preferred_element_type=jnp.float32)
         + jnp.dot(hbn_ref[...], wh_ref[...],
                   preferred_element_type=jnp.float32)
         + b0_ref[...])
    z = _bn_relu(z, g0_ref[...], bb0_ref[...])
    z = _bn_relu(jnp.dot(z, w1_ref[...], preferred_element_type=jnp.float32)
                 + b1_ref[...], g1_ref[...], bb1_ref[...])
    z = _bn_relu(jnp.dot(z, w2_ref[...], preferred_element_type=jnp.float32)
                 + b2_ref[...], g2_ref[...], bb2_ref[...])
    out_ref[...] = (jnp.dot(z, w3_ref[...], preferred_element_type=jnp.float32)
                    + b3_ref[...])


def _d2(feat, gf, bf, hbn, wf, wh, p):
    return pl.pallas_call(
        _d2_body,
        out_shape=jax.ShapeDtypeStruct((_NG, 2), jnp.float32),
    )(feat, gf, bf, hbn, wf, wh,
      p["mlp0_b"].reshape(1, -1), p["mbn0_g"].reshape(1, -1),
      p["mbn0_b"].reshape(1, -1),
      p["mlp1_W"], p["mlp1_b"].reshape(1, -1), p["mbn1_g"].reshape(1, -1),
      p["mbn1_b"].reshape(1, -1),
      p["mlp2_W"], p["mlp2_b"].reshape(1, -1), p["mbn2_g"].reshape(1, -1),
      p["mbn2_b"].reshape(1, -1),
      p["mlp3_W"], p["mlp3_b"].reshape(1, -1))


def kernel(x, edge_index, batch, params):
    del batch
    p = params
    dst_flat = edge_index[1]
    src_r = (edge_index[0] * 2).reshape(_NW * _NCHUNK, _CH)
    dst_r = dst_flat.reshape(_NW * _NCHUNK, _CH)
    zeros_tab = jnp.zeros((_N, _HID), jnp.float32)

    d32 = _deg_call()(dst_flat)
    g0, dinv = _ga(x, p["conv0_W"], d32.T)
    m0 = _scat_call()(g0.reshape(2 * _N, _HID), src_r, dst_r, zeros_tab)
    m0 = m0.reshape(_NC, _N, 2 * _HID)
    h1, g1 = _c(m0, g0, dinv, p["conv0_b"].reshape(1, -1), p["conv1_W"])
    m1 = _scat_call()(g1.reshape(2 * _N, _HID), src_r, dst_r, zeros_tab)
    m1 = m1.reshape(_NC, _N, 2 * _HID)
    feat = _feat_call()(x.reshape(_NG, _F * _F), jnp.asarray(_TRIU_FLAT), m1)
    hbn = _d1(m1, g1, dinv, p["conv1_b"].reshape(1, -1), h1,
              p["bnh_g"].reshape(1, -1), p["bnh_b"].reshape(1, -1))

    wf = p["mlp0_W"][:_TRI]
    wh = p["mlp0_W"][_TRI:]
    return _d2(feat, p["bn_g"].reshape(1, -1), p["bn_b"].reshape(1, -1),
               hbn, wf, wh, p)


# speedup vs baseline: 46.2947x; 1.0406x over previous
import functools

import numpy as np
import jax
import jax.numpy as jnp
from jax import lax
from jax.experimental import pallas as pl
from jax.experimental.pallas import tpu as pltpu
from jax.experimental.pallas import tpu_sc as plsc

_NG = 100
_F = 128
_N = _NG * _F
_E = 409600
_HID = 64
_NC, _NS, _L = 2, 16, 16
_NW = _NC * _NS
_EW = _E // _NW
_CH = 128
_NCHUNK = _EW // _CH
_RPS = _N // _NS
_NBUF = 5

@functools.cache
def _sc_mesh():
    return plsc.VectorSubcoreMesh(core_axis_name="c", subcore_axis_name="s",
                                  num_cores=_NC, num_subcores=_NS)

_IU = np.triu_indices(_F, 1)
_TRIU_FLAT = (_IU[0] * _F + _IU[1]).astype(np.int32)
_TRI = _TRIU_FLAT.size
_GPW = -(-_NG // _NW)


def _deg_body(edge_hbm, out_hbm, idx_v, hist_v):
    c = lax.axis_index("c")
    s = lax.axis_index("s")
    wid = s * _NC + c
    pltpu.sync_copy(edge_hbm.at[1, pl.ds(wid * _EW, _EW)], idx_v)
    zero16 = jnp.zeros((_L,), jnp.float32)
    one16 = jnp.ones((_L,), jnp.float32)

    def zbody(i, carry):
        hist_v[pl.ds(i * _L, _L)] = zero16
        return carry

    lax.fori_loop(0, _N // _L, zbody, 0)

    def hbody(i, carry):
        idx = idx_v[pl.ds(i * _L, _L)]
        plsc.addupdate_scatter(hist_v, [idx], one16)
        return carry

    lax.fori_loop(0, _EW // _L, hbody, 0)
    pltpu.sync_copy(hist_v, out_hbm.at[wid])


@functools.cache
def _deg_call():
    return pl.kernel(
        _deg_body,
        out_type=jax.ShapeDtypeStruct((_NW, _N), jnp.float32),
        mesh=_sc_mesh(),
        scratch_types=[
            pltpu.VMEM((_EW,), jnp.int32),
            pltpu.VMEM((_N,), jnp.float32),
        ],
        compiler_params=pltpu.CompilerParams(needs_layout_passes=False),
    )


def _scat_body(tab_hbm, src_hbm, dst_hbm, zero_hbm, out_hbm,
               sidx_v, didx_v, row_v, widx_v, acc_sh,
               gsems, ssems):
    c = lax.axis_index("c")
    s = lax.axis_index("s")
    wid = s * _NC + c
    pltpu.sync_copy(zero_hbm.at[pl.ds(s * _RPS, _RPS)],
                    acc_sh.at[pl.ds(s * _RPS, _RPS)])
    iota16 = lax.broadcasted_iota(jnp.int32, (_L,), 0)
    for t in range(8):
        for off in (0, 16, 32, 48, 64, 80, 84):
            vals = (iota16 + (s * _RPS + t * 100 + off)) * 2
            widx_v[t, pl.ds(off, _L)] = vals
    pltpu.sync_copy(src_hbm.at[pl.ds(wid * _NCHUNK, _NCHUNK)], sidx_v)
    pltpu.sync_copy(dst_hbm.at[pl.ds(wid * _NCHUNK, _NCHUNK)], didx_v)
    plsc.subcore_barrier()

    def _gather(j, b):
        pltpu.async_copy(tab_hbm.at[sidx_v.at[j]], row_v.at[b], gsems.at[b])

    def _gwait(j, b):
        pltpu.make_async_copy(tab_hbm.at[sidx_v.at[j]], row_v.at[b],
                              gsems.at[b]).wait()

    def _scat(j, b):
        pltpu.async_copy(row_v.at[b], acc_sh.at[didx_v.at[j]], ssems.at[b],
                         add=True)

    def _swait(j, b):
        pltpu.make_async_copy(row_v.at[b], acc_sh.at[didx_v.at[j]],
                              ssems.at[b]).wait()

    half = _NBUF // 2

    def step(j, b):
        @pl.when(j >= _NBUF)
        def _():
            _swait(j - _NBUF, b)

        _gather(j, b)

        @pl.when(j >= half)
        def _():
            _gwait(j - half, (b - half) % _NBUF)
            _scat(j - half, (b - half) % _NBUF)

    def body(i, carry):
        for b in range(_NBUF):
            step(_NBUF * i + b, b)
        return carry

    lax.fori_loop(0, _NCHUNK // _NBUF, body, 0)
    for j in range(_NCHUNK - half, _NCHUNK):
        _gwait(j, j % _NBUF)
        _scat(j, j % _NBUF)
    for j in range(_NCHUNK - _NBUF, _NCHUNK):
        _swait(j, j % _NBUF)
    plsc.subcore_barrier()
    for t in range(8):
        bounce = row_v.at[t % _NBUF, pl.ds(0, 100)]
        pltpu.sync_copy(acc_sh.at[pl.ds(s * _RPS + t * 100, 100)], bounce)
        pltpu.sync_copy(bounce, out_hbm.at[c].at[widx_v.at[t]])


@functools.cache
def _scat_call():
    return pl.kernel(
        _scat_body,
        out_type=jax.ShapeDtypeStruct((_NC, 2 * _N, _HID), jnp.float32),
        mesh=_sc_mesh(),
        scratch_types=[
            pltpu.VMEM((_NCHUNK, _CH), jnp.int32),
            pltpu.VMEM((_NCHUNK, _CH), jnp.int32),
            pltpu.VMEM((_NBUF, _CH, _HID), jnp.float32),
            pltpu.VMEM((8, 100), jnp.int32),
            pltpu.VMEM_SHARED((_N, _HID), jnp.float32),
            pltpu.SemaphoreType.DMA((_NBUF,)),
            pltpu.SemaphoreType.DMA((_NBUF,)),
        ],
        compiler_params=pltpu.CompilerParams(needs_layout_passes=False,
                                             use_tc_tiling_on_sc=False),
    )


def _feat_body(x_hbm, tri_hbm, dep_hbm, out_hbm, xrow_v, tri_v, feat_v,
               isems, osems):
    del dep_hbm
    c = lax.axis_index("c")
    s = lax.axis_index("s")
    wid = s * _NC + c
    pltpu.sync_copy(tri_hbm, tri_v)

    def _load(k):
        g = wid + _NW * k

        @pl.when(g < _NG)
        def _():
            pltpu.async_copy(x_hbm.at[g], xrow_v.at[k % 2], isems.at[k % 2])

    _load(0)
    for k in range(_GPW):
        g = wid + _NW * k
        if k + 1 < _GPW:
            _load(k + 1)

        @pl.when(g < _NG)
        def _():
            pltpu.make_async_copy(x_hbm.at[g], xrow_v.at[k % 2],
                                  isems.at[k % 2]).wait()
            row = xrow_v.at[k % 2]
            fv = feat_v.at[k % 2]

            def gbody(i, carry):
                for u in range(4):
                    o = i * 4 * _L + u * _L
                    fv[pl.ds(o, _L)] = plsc.load_gather(row,
                                                       [tri_v[pl.ds(o, _L)]])
                return carry

            lax.fori_loop(0, _TRI // (4 * _L), gbody, 0)
            if k >= 2:
                pltpu.make_async_copy(
                    fv, out_hbm.at[g - 2 * _NW, pl.ds(0, _TRI)],
                    osems.at[k % 2]).wait()
            pltpu.async_copy(fv, out_hbm.at[g, pl.ds(0, _TRI)],
                             osems.at[k % 2])
    for k in range(_GPW):
        g = wid + _NW * k
        if k + 2 < _GPW:
            cond = jnp.logical_and(g < _NG, wid + _NW * (k + 2) >= _NG)
        else:
            cond = g < _NG

        @pl.when(cond)
        def _():
            pltpu.make_async_copy(feat_v.at[k % 2],
                                  out_hbm.at[g, pl.ds(0, _TRI)],
                                  osems.at[k % 2]).wait()


@functools.cache
def _feat_call():
    return pl.kernel(
        _feat_body,
        out_type=jax.ShapeDtypeStruct((_NG, 8192), jnp.float32),
        mesh=_sc_mesh(),
        scratch_types=[
            pltpu.VMEM((2, _F * _F), jnp.float32),
            pltpu.VMEM((_TRI,), jnp.int32),
            pltpu.VMEM((2, _TRI), jnp.float32),
            pltpu.SemaphoreType.DMA((2,)),
            pltpu.SemaphoreType.DMA((2,)),
        ],
        compiler_params=pltpu.CompilerParams(needs_layout_passes=False,
                                             use_tc_tiling_on_sc=False),
    )


def _ga_body(x_ref, w0_ref, d32_ref, g0_ref, dinv_ref):
    deg = jnp.sum(d32_ref[...], axis=1, keepdims=True) + 1.0
    dinv = lax.rsqrt(jnp.maximum(deg, 1.0))
    hw = jnp.dot(x_ref[...], w0_ref[...], preferred_element_type=jnp.float32)
    g0_ref[...] = jnp.concatenate(
        [hw * dinv, jnp.zeros((_N, _HID), jnp.float32)], axis=1)
    dinv_ref[...] = dinv


def _ga(x, w0, d32t):
    return pl.pallas_call(
        _ga_body,
        out_shape=(
            jax.ShapeDtypeStruct((_N, 2 * _HID), jnp.float32),
            jax.ShapeDtypeStruct((_N, 1), jnp.float32),
        ),
    )(x, w0, d32t)


def _c_body(m_ref, g0_ref, dinv_ref, b0_ref, w1_ref, h1_ref, g1_ref):
    dinv = dinv_ref[...]
    m = m_ref[0, :, :_HID] + m_ref[1, :, :_HID] + g0_ref[:, :_HID]
    h1 = jnp.tanh(m * dinv + b0_ref[...])
    h1_ref[...] = h1
    g1 = jnp.dot(h1, w1_ref[...], preferred_element_type=jnp.float32) * dinv
    g1_ref[...] = jnp.concatenate(
        [g1, jnp.zeros((_N, _HID), jnp.float32)], axis=1)


def _c(m0, g0, dinv, b0, w1):
    return pl.pallas_call(
        _c_body,
        out_shape=(
            jax.ShapeDtypeStruct((_N, _HID), jnp.float32),
            jax.ShapeDtypeStruct((_N, 2 * _HID), jnp.float32),
        ),
    )(m0, g0, dinv, b0, w1)


def _d1_body(m_ref, g1_ref, dinv_ref, b1_ref, h1_ref, bg_ref, bb_ref, out_ref):
    dinv = dinv_ref[...]
    m = m_ref[0, :, :_HID] + m_ref[1, :, :_HID] + g1_ref[:, :_HID]
    h2 = jnp.tanh(m * dinv + b1_ref[...])
    hcat = jnp.concatenate([h1_ref[...], h2], axis=1)
    rr = lax.broadcasted_iota(jnp.int32, (_NG, _N), 0)
    cc = lax.broadcasted_iota(jnp.int32, (_NG, _N), 1)
    sel = jnp.where((cc // _F) == rr, jnp.float32(1.0 / _F), jnp.float32(0.0))
    m = jnp.dot(sel, hcat, preferred_element_type=jnp.float32)
    mu = jnp.mean(m, axis=0, keepdims=True)
    var = jnp.mean((m - mu) ** 2, axis=0, keepdims=True)
    out_ref[...] = (m - mu) * lax.rsqrt(var + 1e-5) * bg_ref[...] + bb_ref[...]


def _d1(m1, g1, dinv, b1, h1, bg, bb):
    return pl.pallas_call(
        _d1_body,
        out_shape=jax.ShapeDtypeStruct((_NG, _F), jnp.float32),
    )(m1, g1, dinv, b1, h1, bg, bb)


def _bn_relu(z, g, b):
    mu = jnp.mean(z, axis=0, keepdims=True)
    var = jnp.mean((z - mu) ** 2, axis=0, keepdims=True)
    return jnp.maximum((z - mu) * lax.rsqrt(var + 1e-5) * g + b,
                       jnp.float32(0.0))


def _d2_body(feat_ref, gf_ref, bf_ref, hbn_ref, wf_ref, wh_ref,
             b0_ref, g0_ref, bb0_ref, w1_ref, b1_ref, g1_ref, bb1_ref,
             w2_ref, b2_ref, g2_ref, bb2_ref, w3_ref, b3_ref, out_ref):
    f = feat_ref[:, :_TRI]
    mu = jnp.mean(f, axis=0, keepdims=True)
    var = jnp.mean((f - mu) ** 2, axis=0, keepdims=True)
    fbn = (f - mu) * lax.rsqrt(var + 1e-5) * gf_ref[...] + bf_ref[...]
    z = (jnp.dot(fbn, wf_ref[...], preferred_element_type=jnp.float32)
         + jnp.dot(hbn_ref[...], wh_ref[...],
                   preferred_element_type=jnp.float32)
         + b0_ref[...])
    z = _bn_relu(z, g0_ref[...], bb0_ref[...])
    z = _bn_relu(jnp.dot(z, w1_ref[...], preferred_element_type=jnp.float32)
                 + b1_ref[...], g1_ref[...], bb1_ref[...])
    z = _bn_relu(jnp.dot(z, w2_ref[...], preferred_element_type=jnp.float32)
                 + b2_ref[...], g2_ref[...], bb2_ref[...])
    out_ref[...] = (jnp.dot(z, w3_ref[...], preferred_element_type=jnp.float32)
                    + b3_ref[...])


def _d2(feat, gf, bf, hbn, wf, wh, p):
    return pl.pallas_call(
        _d2_body,
        out_shape=jax.ShapeDtypeStruct((_NG, 2), jnp.float32),
    )(feat, gf, bf, hbn, wf, wh,
      p["mlp0_b"].reshape(1, -1), p["mbn0_g"].reshape(1, -1),
      p["mbn0_b"].reshape(1, -1),
      p["mlp1_W"], p["mlp1_b"].reshape(1, -1), p["mbn1_g"].reshape(1, -1),
      p["mbn1_b"].reshape(1, -1),
      p["mlp2_W"], p["mlp2_b"].reshape(1, -1), p["mbn2_g"].reshape(1, -1),
      p["mbn2_b"].reshape(1, -1),
      p["mlp3_W"], p["mlp3_b"].reshape(1, -1))


def kernel(x, edge_index, batch, params):
    del batch
    p = params
    dst_flat = edge_index[1]
    src_r = (edge_index[0] * 2).reshape(_NW * _NCHUNK, _CH)
    dst_r = dst_flat.reshape(_NW * _NCHUNK, _CH)
    zeros_tab = jnp.zeros((_N, _HID), jnp.float32)

    d32 = _deg_call()(edge_index)
    g0, dinv = _ga(x, p["conv0_W"], d32.T)
    m0 = _scat_call()(g0.reshape(2 * _N, _HID), src_r, dst_r, zeros_tab)
    m0 = m0.reshape(_NC, _N, 2 * _HID)
    h1, g1 = _c(m0, g0, dinv, p["conv0_b"].reshape(1, -1), p["conv1_W"])
    m1 = _scat_call()(g1.reshape(2 * _N, _HID), src_r, dst_r, zeros_tab)
    m1 = m1.reshape(_NC, _N, 2 * _HID)
    feat = _feat_call()(x.reshape(_NG, _F * _F), jnp.asarray(_TRIU_FLAT), m1)
    hbn = _d1(m1, g1, dinv, p["conv1_b"].reshape(1, -1), h1,
              p["bnh_g"].reshape(1, -1), p["bnh_b"].reshape(1, -1))

    wf = p["mlp0_W"][:_TRI]
    wh = p["mlp0_W"][_TRI:]
    return _d2(feat, p["bn_g"].reshape(1, -1), p["bn_b"].reshape(1, -1),
               hbn, wf, wh, p)
